# baseline (speedup 1.0000x reference)
import jax
import jax.numpy as jnp
from jax import lax
from jax.experimental import pallas as pl
from jax.experimental.pallas import tpu as pltpu
from jax.experimental.pallas import tpu_sc as plsc

N_VOCAB = 100000
D_MODEL = 1024
BATCH = 4
SEQ = 4096
B_FLAT = BATCH * SEQ
LANES = 16
NSEG = D_MODEL // LANES

NC = 2
NS = 16
NW = NC * NS
POS_PW = SEQ // NW
P_CH = 8
NCH = POS_PW // P_CH
NBUF = 2


def _embed_pe_kernel(idx_hbm, table_hbm, pe_hbm, out_hbm,
                     idx_all, pe0, pe1, rows0, rows1,
                     idx_sem, pe_sems, in_sems, out_sems):
    c_id = lax.axis_index("c")
    s_id = lax.axis_index("s")
    wid = s_id * NC + c_id
    pos_base = wid * POS_PW

    pe_b = (pe0, pe1)
    rows_b = (rows0, rows1)

    for b in range(BATCH):
        pltpu.async_copy(idx_hbm.at[b, pl.ds(pos_base, POS_PW)], idx_all.at[b], idx_sem)
    for b in range(BATCH):
        pltpu.make_async_copy(idx_hbm.at[b, pl.ds(pos_base, POS_PW)], idx_all.at[b], idx_sem).wait()

    def fetch(c, slot):
        pos0 = pos_base + c * P_CH
        pltpu.async_copy(pe_hbm.at[pl.ds(pos0, P_CH)], pe_b[slot], pe_sems.at[slot])
        for b in range(BATCH):
            pltpu.async_copy(
                table_hbm.at[idx_all.at[b, pl.ds(c * P_CH, P_CH)]],
                rows_b[slot].at[pl.ds(b * P_CH, P_CH)],
                in_sems.at[slot, b],
            )

    def wait_store(c, slot):
        pos0 = pos_base + c * P_CH
        for b in range(BATCH):
            pltpu.make_async_copy(
                rows_b[slot].at[pl.ds(b * P_CH, P_CH)],
                out_hbm.at[pl.ds(b * SEQ + pos0, P_CH)],
                out_sems.at[slot],
            ).wait()

    def step(c, slot):
        pos0 = pos_base + c * P_CH
        rows = rows_b[slot]
        pe_buf = pe_b[slot]

        pltpu.make_async_copy(pe_hbm.at[pl.ds(pos0, P_CH)], pe_buf, pe_sems.at[slot]).wait()
        for b in range(BATCH):
            pltpu.make_async_copy(
                table_hbm.at[idx_all.at[b, pl.ds(c * P_CH, P_CH)]],
                rows.at[pl.ds(b * P_CH, P_CH)],
                in_sems.at[slot, b],
            ).wait()

            def seg_body(j, carry, b=b):
                col = j * LANES
                for p in range(P_CH):
                    pv = pe_buf[p, pl.ds(col, LANES)]
                    r = b * P_CH + p
                    rows[r, pl.ds(col, LANES)] = rows[r, pl.ds(col, LANES)] + pv
                return carry

            lax.fori_loop(0, NSEG, seg_body, 0)
            pltpu.async_copy(
                rows.at[pl.ds(b * P_CH, P_CH)],
                out_hbm.at[pl.ds(b * SEQ + pos0, P_CH)],
                out_sems.at[slot],
            )

        @pl.when(c + 1 < NCH)
        def _():
            @pl.when(c >= 1)
            def _():
                wait_store(c - 1, (slot + 1) % NBUF)
            fetch(c + 1, (slot + 1) % NBUF)

    fetch(0, 0)

    def outer(i, carry):
        for b in range(NBUF):
            step(i * NBUF + b, b)
        return carry

    lax.fori_loop(0, NCH // NBUF, outer, 0)
    wait_store(NCH - 2, (NCH - 2) % NBUF)
    wait_store(NCH - 1, (NCH - 1) % NBUF)


@jax.jit
def _run(idx, table, pe):
    mesh = plsc.VectorSubcoreMesh(core_axis_name="c", subcore_axis_name="s")
    k = pl.kernel(
        _embed_pe_kernel,
        mesh=mesh,
        out_type=jax.ShapeDtypeStruct((B_FLAT, D_MODEL), jnp.float32),
        scratch_types=[
            pltpu.VMEM((BATCH, POS_PW), jnp.int32),
            pltpu.VMEM((P_CH, D_MODEL), jnp.float32),
            pltpu.VMEM((P_CH, D_MODEL), jnp.float32),
            pltpu.VMEM((BATCH * P_CH, D_MODEL), jnp.float32),
            pltpu.VMEM((BATCH * P_CH, D_MODEL), jnp.float32),
            pltpu.SemaphoreType.DMA,
            pltpu.SemaphoreType.DMA((NBUF,)),
            pltpu.SemaphoreType.DMA((NBUF, BATCH)),
            pltpu.SemaphoreType.DMA((NBUF,)),
        ],
    )
    return k(idx, table, pe)


def kernel(input, embed_table, pe):
    out = _run(input.astype(jnp.int32), embed_table, pe)
    return out.reshape(BATCH, SEQ, D_MODEL)


# speedup vs baseline: 1.3487x; 1.3487x over previous
import jax
import jax.numpy as jnp
from jax import lax
from jax.experimental import pallas as pl
from jax.experimental.pallas import tpu as pltpu
from jax.experimental.pallas import tpu_sc as plsc

N_VOCAB = 100000
D_MODEL = 1024
BATCH = 4
SEQ = 4096
B_FLAT = BATCH * SEQ
LANES = 16
NSEG = D_MODEL // LANES

NC = 2
NS = 16
NW = NC * NS
POS_PW = SEQ // NW
P_CH = 8
NCH = POS_PW // P_CH
NBUF = 3


def _embed_pe_kernel(idx_hbm, table_hbm, pe_hbm, out_hbm,
                     idx_all, pe0, pe1, pe2, rows0, rows1, rows2,
                     idx_sem, pe_sems, in_sems, out_sems):
    c_id = lax.axis_index("c")
    s_id = lax.axis_index("s")
    wid = s_id * NC + c_id
    pos_base = wid * POS_PW

    pe_b = (pe0, pe1, pe2)
    rows_b = (rows0, rows1, rows2)

    for b in range(BATCH):
        pltpu.async_copy(idx_hbm.at[b, pl.ds(pos_base, POS_PW)], idx_all.at[b], idx_sem)
    for b in range(BATCH):
        pltpu.make_async_copy(idx_hbm.at[b, pl.ds(pos_base, POS_PW)], idx_all.at[b], idx_sem).wait()

    def fetch(c, slot):
        pos0 = pos_base + c * P_CH
        pltpu.async_copy(pe_hbm.at[pl.ds(pos0, P_CH)], pe_b[slot], pe_sems.at[slot])
        for b in range(BATCH):
            pltpu.async_copy(
                table_hbm.at[idx_all.at[b, pl.ds(c * P_CH, P_CH)]],
                rows_b[slot].at[pl.ds(b * P_CH, P_CH)],
                in_sems.at[slot, b],
            )

    def wait_store(c, slot):
        pos0 = pos_base + c * P_CH
        for b in range(BATCH):
            pltpu.make_async_copy(
                rows_b[slot].at[pl.ds(b * P_CH, P_CH)],
                out_hbm.at[pl.ds(b * SEQ + pos0, P_CH)],
                out_sems.at[slot],
            ).wait()

    def step(c, slot):
        pos0 = pos_base + c * P_CH
        rows = rows_b[slot]
        pe_buf = pe_b[slot]

        pltpu.make_async_copy(pe_hbm.at[pl.ds(pos0, P_CH)], pe_buf, pe_sems.at[slot]).wait()
        for b in range(BATCH):
            pltpu.make_async_copy(
                table_hbm.at[idx_all.at[b, pl.ds(c * P_CH, P_CH)]],
                rows.at[pl.ds(b * P_CH, P_CH)],
                in_sems.at[slot, b],
            ).wait()

            def seg_body(j, carry, b=b):
                col = j * LANES
                for p in range(P_CH):
                    pv = pe_buf[p, pl.ds(col, LANES)]
                    r = b * P_CH + p
                    rows[r, pl.ds(col, LANES)] = rows[r, pl.ds(col, LANES)] + pv
                return carry

            lax.fori_loop(0, NSEG, seg_body, 0)
            pltpu.async_copy(
                rows.at[pl.ds(b * P_CH, P_CH)],
                out_hbm.at[pl.ds(b * SEQ + pos0, P_CH)],
                out_sems.at[slot],
            )

            if b == BATCH - 2:
                @pl.when(c + 2 < NCH)
                def _():
                    @pl.when(c >= 1)
                    def _():
                        wait_store(c - 1, (slot + 2) % NBUF)
                    fetch(c + 2, (slot + 2) % NBUF)

    fetch(0, 0)
    fetch(1, 1)

    def outer(i, carry):
        for b in range(NBUF):
            step(i * NBUF + b, b)
        return carry

    lax.fori_loop(0, (NCH - 1) // NBUF, outer, 0)
    step(NCH - 1, (NCH - 1) % NBUF)
    wait_store(NCH - 3, (NCH - 3) % NBUF)
    wait_store(NCH - 2, (NCH - 2) % NBUF)
    wait_store(NCH - 1, (NCH - 1) % NBUF)


@jax.jit
def _run(idx, table, pe):
    mesh = plsc.VectorSubcoreMesh(core_axis_name="c", subcore_axis_name="s")
    k = pl.kernel(
        _embed_pe_kernel,
        mesh=mesh,
        out_type=jax.ShapeDtypeStruct((B_FLAT, D_MODEL), jnp.float32),
        scratch_types=[
            pltpu.VMEM((BATCH, POS_PW), jnp.int32),
            pltpu.VMEM((P_CH, D_MODEL), jnp.float32),
            pltpu.VMEM((P_CH, D_MODEL), jnp.float32),
            pltpu.VMEM((P_CH, D_MODEL), jnp.float32),
            pltpu.VMEM((BATCH * P_CH, D_MODEL), jnp.float32),
            pltpu.VMEM((BATCH * P_CH, D_MODEL), jnp.float32),
            pltpu.VMEM((BATCH * P_CH, D_MODEL), jnp.float32),
            pltpu.SemaphoreType.DMA,
            pltpu.SemaphoreType.DMA((NBUF,)),
            pltpu.SemaphoreType.DMA((NBUF, BATCH)),
            pltpu.SemaphoreType.DMA((NBUF,)),
        ],
    )
    return k(idx, table, pe)


def kernel(input, embed_table, pe):
    out = _run(input.astype(jnp.int32), embed_table, pe)
    return out.reshape(BATCH, SEQ, D_MODEL)


# speedup vs baseline: 1.3500x; 1.0010x over previous
import jax
import jax.numpy as jnp
from jax import lax
from jax.experimental import pallas as pl
from jax.experimental.pallas import tpu as pltpu
from jax.experimental.pallas import tpu_sc as plsc

N_VOCAB = 100000
D_MODEL = 1024
BATCH = 4
SEQ = 4096
B_FLAT = BATCH * SEQ
LANES = 16
NSEG = D_MODEL // LANES

NC = 2
NS = 16
NW = NC * NS
POS_PW = SEQ // NW
P_CH = 8
NCH = POS_PW // P_CH
NBUF = 3


def _embed_pe_kernel(idx_hbm, table_hbm, pe_hbm, out_hbm,
                     idx_all, pe0, pe1, pe2, rows0, rows1, rows2,
                     idx_sem, pe_sems, in_sems, out_sems):
    c_id = lax.axis_index("c")
    s_id = lax.axis_index("s")
    wid = s_id * NC + c_id
    pos_base = wid * POS_PW

    pe_b = (pe0, pe1, pe2)
    rows_b = (rows0, rows1, rows2)

    for b in range(BATCH):
        pltpu.async_copy(idx_hbm.at[b, pl.ds(pos_base, POS_PW)], idx_all.at[b], idx_sem)
    for b in range(BATCH):
        pltpu.make_async_copy(idx_hbm.at[b, pl.ds(pos_base, POS_PW)], idx_all.at[b], idx_sem).wait()

    def fetch(c, slot):
        pos0 = pos_base + c * P_CH
        pltpu.async_copy(pe_hbm.at[pl.ds(pos0, P_CH)], pe_b[slot], pe_sems.at[slot])
        for b in range(BATCH):
            pltpu.async_copy(
                table_hbm.at[idx_all.at[b, pl.ds(c * P_CH, P_CH)]],
                rows_b[slot].at[pl.ds(b * P_CH, P_CH)],
                in_sems.at[slot, b],
            )

    def wait_store(c, slot):
        pos0 = pos_base + c * P_CH
        for b in range(BATCH):
            pltpu.make_async_copy(
                rows_b[slot].at[pl.ds(b * P_CH, P_CH)],
                out_hbm.at[pl.ds(b * SEQ + pos0, P_CH)],
                out_sems.at[slot],
            ).wait()

    def step(c, slot):
        pos0 = pos_base + c * P_CH
        rows = rows_b[slot]
        pe_buf = pe_b[slot]

        pltpu.make_async_copy(pe_hbm.at[pl.ds(pos0, P_CH)], pe_buf, pe_sems.at[slot]).wait()
        for b in range(BATCH):
            pltpu.make_async_copy(
                table_hbm.at[idx_all.at[b, pl.ds(c * P_CH, P_CH)]],
                rows.at[pl.ds(b * P_CH, P_CH)],
                in_sems.at[slot, b],
            ).wait()

            def seg_body(j, carry, b=b):
                col = j * LANES
                for p in range(P_CH):
                    pv = pe_buf[p, pl.ds(col, LANES)]
                    r = b * P_CH + p
                    rows[r, pl.ds(col, LANES)] = rows[r, pl.ds(col, LANES)] + pv
                return carry

            lax.fori_loop(0, NSEG, seg_body, 0)
            pltpu.async_copy(
                rows.at[pl.ds(b * P_CH, P_CH)],
                out_hbm.at[pl.ds(b * SEQ + pos0, P_CH)],
                out_sems.at[slot],
            )

            if b == BATCH - 3:
                @pl.when(c + 2 < NCH)
                def _():
                    @pl.when(c >= 1)
                    def _():
                        wait_store(c - 1, (slot + 2) % NBUF)
                    fetch(c + 2, (slot + 2) % NBUF)

    fetch(0, 0)
    fetch(1, 1)

    def outer(i, carry):
        for b in range(NBUF):
            step(i * NBUF + b, b)
        return carry

    lax.fori_loop(0, (NCH - 1) // NBUF, outer, 0)
    step(NCH - 1, (NCH - 1) % NBUF)
    wait_store(NCH - 3, (NCH - 3) % NBUF)
    wait_store(NCH - 2, (NCH - 2) % NBUF)
    wait_store(NCH - 1, (NCH - 1) % NBUF)


@jax.jit
def _run(idx, table, pe):
    mesh = plsc.VectorSubcoreMesh(core_axis_name="c", subcore_axis_name="s")
    k = pl.kernel(
        _embed_pe_kernel,
        mesh=mesh,
        out_type=jax.ShapeDtypeStruct((B_FLAT, D_MODEL), jnp.float32),
        scratch_types=[
            pltpu.VMEM((BATCH, POS_PW), jnp.int32),
            pltpu.VMEM((P_CH, D_MODEL), jnp.float32),
            pltpu.VMEM((P_CH, D_MODEL), jnp.float32),
            pltpu.VMEM((P_CH, D_MODEL), jnp.float32),
            pltpu.VMEM((BATCH * P_CH, D_MODEL), jnp.float32),
            pltpu.VMEM((BATCH * P_CH, D_MODEL), jnp.float32),
            pltpu.VMEM((BATCH * P_CH, D_MODEL), jnp.float32),
            pltpu.SemaphoreType.DMA,
            pltpu.SemaphoreType.DMA((NBUF,)),
            pltpu.SemaphoreType.DMA((NBUF, BATCH)),
            pltpu.SemaphoreType.DMA((NBUF,)),
        ],
    )
    return k(idx, table, pe)


def kernel(input, embed_table, pe):
    out = _run(input.astype(jnp.int32), embed_table, pe)
    return out.reshape(BATCH, SEQ, D_MODEL)
